# Initial kernel scaffold; baseline (speedup 1.0000x reference)
import jax
import jax.numpy as jnp
from jax.experimental import pallas as pl


def kernel(x, pe_weight):
    raise NotImplementedError("write your pallas kernel here")


# speedup vs baseline: 1.0258x; 1.0258x over previous
import jax
import jax.numpy as jnp
from jax.experimental import pallas as pl


def _body(x_ref, pe_ref, o_ref):
    o_ref[...] = x_ref[...] + pe_ref[...][None]


def kernel(x, pe_weight):
    n_i, n_j, d = x.shape
    bi = 16
    return pl.pallas_call(
        _body,
        grid=(n_i // bi,),
        in_specs=[
            pl.BlockSpec((bi, n_j, d), lambda i: (i, 0, 0)),
            pl.BlockSpec((n_j, d), lambda i: (0, 0)),
        ],
        out_specs=pl.BlockSpec((bi, n_j, d), lambda i: (i, 0, 0)),
        out_shape=jax.ShapeDtypeStruct((n_i, n_j, d), x.dtype),
    )(x, pe_weight)
